# baseline (speedup 1.0000x reference)
import functools

import jax
import jax.numpy as jnp
from jax import lax
from jax.experimental import pallas as pl
from jax.experimental.pallas import tpu as pltpu
from jax.experimental.pallas import tpu_sc as plsc

VOCAB = 1000000
SEQLEN = 200
EMBED = 64
BATCH = 4096
LANES = 16

NW = 32
BT = BATCH // NW
FT = EMBED // 8


def _make_kernel():
    mesh = plsc.VectorSubcoreMesh(core_axis_name="c", subcore_axis_name="s")

    @functools.partial(
        pl.kernel,
        mesh=mesh,
        out_type=jax.ShapeDtypeStruct((SEQLEN, FT, NW, 8, BT), jnp.float32),
        compiler_params=pltpu.CompilerParams(
            use_tc_tiling_on_sc=False, needs_layout_passes=False
        ),
        scratch_types=[
            pltpu.VMEM((SEQLEN // 8, 8, BT), jnp.int32),
            pltpu.VMEM((FT, 2, 8, BT), jnp.float32),
            pltpu.VMEM((SEQLEN, EMBED), jnp.float32),
        ] + [pltpu.VMEM((BT, EMBED), jnp.float32) for _ in range(4)]
          + [pltpu.VMEM((FT, 8, BT), jnp.float32) for _ in range(4)]
          + [pltpu.SemaphoreType.DMA for _ in range(8)],
    )
    def emb(
        table_hbm, idx_hbm, pos_hbm, out_hbm,
        idx_v, pos_t_v, pos_v,
        rows0, rows1, rows2, rows3,
        slab0, slab1, slab2, slab3,
        gsem0, gsem1, gsem2, gsem3,
        wsem0, wsem1, wsem2, wsem3,
    ):
        w = lax.axis_index("s") * 2 + lax.axis_index("c")
        b0 = pl.multiple_of(w * BT, 8)

        rows = (rows0, rows1, rows2, rows3)
        slab = (slab0, slab1, slab2, slab3)
        gsem = (gsem0, gsem1, gsem2, gsem3)
        wsem = (wsem0, wsem1, wsem2, wsem3)

        pltpu.sync_copy(idx_hbm.at[:, w], idx_v)
        pltpu.sync_copy(pos_hbm, pos_t_v)

        lane0 = lax.iota(jnp.int32, LANES)

        f_hi = [(q * LANES + lane0) >> 3 for q in range(EMBED // LANES)]
        f_lo = [(q * LANES + lane0) & 7 for q in range(EMBED // LANES)]

        def pos_body(s, _):
            st2 = jnp.broadcast_to(s >> 7, (LANES,))
            sc = jnp.broadcast_to(s & 127, (LANES,))
            for q in range(EMBED // LANES):
                vec = plsc.load_gather(pos_t_v, [f_hi[q], st2, f_lo[q], sc])
                pos_v[s, pl.ds(q * LANES, LANES)] = vec
            return 0

        lax.fori_loop(0, SEQLEN, pos_body, 0)

        def gather_copy(s, b):
            return pltpu.make_async_copy(
                table_hbm.at[idx_v.at[s // 8, s % 8]], rows[b], gsem[b]
            )

        def write_copy(s, b):
            return pltpu.make_async_copy(
                slab[b], out_hbm.at[s, :, w], wsem[b]
            )

        lane = lax.iota(jnp.int32, LANES)
        row_ids = [lane + c * LANES for c in range(BT // LANES)]

        def transpose_add(s, b):
            pos_q = [pos_v[s, pl.ds(q * LANES, LANES)] for q in range(4)]

            def d_body(d, _):
                rot = (lane + d) & 15
                cols = [rot + q * LANES for q in range(4)]
                pos_d = [
                    pos_q[q].at[rot].get(mode="promise_in_bounds")
                    for q in range(4)
                ]
                vals = [
                    plsc.load_gather(rows[b], [row_ids[c], cols[q]])
                    for q in range(4)
                    for c in range(BT // LANES)
                ]
                sums = [
                    vals[q * (BT // LANES) + c] + pos_d[q]
                    for q in range(4)
                    for c in range(BT // LANES)
                ]
                for q in range(4):
                    for c in range(BT // LANES):
                        plsc.store_scatter(
                            slab[b],
                            [cols[q] >> 3, cols[q] & 7, row_ids[c]],
                            sums[q * (BT // LANES) + c],
                        )
                return 0

            lax.fori_loop(0, LANES, d_body, 0)

        NB = 4
        NI = SEQLEN // NB

        for b in range(NB):
            gather_copy(b, b).start()

        def loop_body(i, carry):
            for b in range(NB):
                s = i * NB + b
                gather_copy(s, b).wait()

                @pl.when(i > 0)
                def _():
                    write_copy(s - NB, b).wait()
                transpose_add(s, b)
                write_copy(s, b).start()

                @pl.when(i < NI - 1)
                def _():
                    gather_copy(s + NB, b).start()
            return carry

        lax.fori_loop(0, NI, loop_body, 0, unroll=False)

        for b in range(NB):
            write_copy(SEQLEN - NB + b, b).wait()

    return emb


_emb = _make_kernel()


@jax.jit
def kernel(inp, token_table, pos_table):
    idx4 = (
        inp.astype(jnp.int32)
        .T.reshape(SEQLEN // 8, 8, NW, BT)
        .transpose(0, 2, 1, 3)
    )
    pos4 = (
        jnp.pad(pos_table.T, ((0, 0), (0, 56)))
        .reshape(FT, 8, 2, BT)
        .transpose(0, 2, 1, 3)
    )
    out5 = _emb(token_table, idx4, pos4)
    return (
        out5.transpose(2, 4, 0, 1, 3)
        .reshape(BATCH, SEQLEN, EMBED)
    )


# speedup vs baseline: 1.0220x; 1.0220x over previous
import functools

import jax
import jax.numpy as jnp
from jax import lax
from jax.experimental import pallas as pl
from jax.experimental.pallas import tpu as pltpu
from jax.experimental.pallas import tpu_sc as plsc

VOCAB = 1000000
SEQLEN = 200
EMBED = 64
BATCH = 4096
LANES = 16

NW = 32
BT = BATCH // NW
FT = EMBED // 8


def _make_kernel():
    mesh = plsc.VectorSubcoreMesh(core_axis_name="c", subcore_axis_name="s")

    @functools.partial(
        pl.kernel,
        mesh=mesh,
        out_type=jax.ShapeDtypeStruct((SEQLEN, FT, NW, 8, BT), jnp.float32),
        compiler_params=pltpu.CompilerParams(
            use_tc_tiling_on_sc=False, needs_layout_passes=False
        ),
        scratch_types=[
            pltpu.VMEM((SEQLEN // 8, 8, BT), jnp.int32),
            pltpu.VMEM((FT, 2, 8, BT), jnp.float32),
            pltpu.VMEM((SEQLEN, EMBED), jnp.float32),
        ] + [pltpu.VMEM((BT, EMBED), jnp.float32) for _ in range(4)]
          + [pltpu.VMEM((FT, 8, BT), jnp.float32) for _ in range(4)]
          + [pltpu.SemaphoreType.DMA for _ in range(8)],
    )
    def emb(
        table_hbm, idx_hbm, pos_hbm, out_hbm,
        idx_v, pos_t_v, pos_v,
        rows0, rows1, rows2, rows3,
        slab0, slab1, slab2, slab3,
        gsem0, gsem1, gsem2, gsem3,
        wsem0, wsem1, wsem2, wsem3,
    ):
        w = lax.axis_index("s") * 2 + lax.axis_index("c")
        b0 = pl.multiple_of(w * BT, 8)

        rows = (rows0, rows1, rows2, rows3)
        slab = (slab0, slab1, slab2, slab3)
        gsem = (gsem0, gsem1, gsem2, gsem3)
        wsem = (wsem0, wsem1, wsem2, wsem3)

        pltpu.sync_copy(idx_hbm.at[:, w], idx_v)

        lane0 = lax.iota(jnp.int32, LANES)

        f_hi = [(q * LANES + lane0) >> 3 for q in range(EMBED // LANES)]
        f_lo = [(q * LANES + lane0) & 7 for q in range(EMBED // LANES)]

        def pos_body(s, _):
            st2 = jnp.broadcast_to(s >> 7, (LANES,))
            sc = jnp.broadcast_to(s & 127, (LANES,))
            for q in range(EMBED // LANES):
                vec = plsc.load_gather(pos_t_v, [f_hi[q], st2, f_lo[q], sc])
                pos_v[s, pl.ds(q * LANES, LANES)] = vec
            return 0

        def pos_prologue():
            pltpu.sync_copy(pos_hbm, pos_t_v)
            lax.fori_loop(0, SEQLEN, pos_body, 0)

        def gather_copy(s, b):
            return pltpu.make_async_copy(
                table_hbm.at[idx_v.at[s // 8, s % 8]], rows[b], gsem[b]
            )

        def write_copy(s, b):
            return pltpu.make_async_copy(
                slab[b], out_hbm.at[s, :, w], wsem[b]
            )

        lane = lax.iota(jnp.int32, LANES)
        row_ids = [lane + c * LANES for c in range(BT // LANES)]

        def transpose_add(s, b):
            pos_q = [pos_v[s, pl.ds(q * LANES, LANES)] for q in range(4)]

            def d_body(d, _):
                rot = (lane + d) & 15
                cols = [rot + q * LANES for q in range(4)]
                pos_d = [
                    pos_q[q].at[rot].get(mode="promise_in_bounds")
                    for q in range(4)
                ]
                vals = [
                    plsc.load_gather(rows[b], [row_ids[c], cols[q]])
                    for q in range(4)
                    for c in range(BT // LANES)
                ]
                sums = [
                    vals[q * (BT // LANES) + c] + pos_d[q]
                    for q in range(4)
                    for c in range(BT // LANES)
                ]
                for q in range(4):
                    for c in range(BT // LANES):
                        plsc.store_scatter(
                            slab[b],
                            [cols[q] >> 3, cols[q] & 7, row_ids[c]],
                            sums[q * (BT // LANES) + c],
                        )
                return 0

            lax.fori_loop(0, LANES, d_body, 0, unroll=2)

        NB = 4
        NI = SEQLEN // NB

        for b in range(NB):
            gather_copy(b, b).start()
        pos_prologue()

        def loop_body(i, carry):
            for b in range(NB):
                s = i * NB + b
                gather_copy(s, b).wait()

                @pl.when(i > 0)
                def _():
                    write_copy(s - NB, b).wait()
                transpose_add(s, b)
                write_copy(s, b).start()

                @pl.when(i < NI - 1)
                def _():
                    gather_copy(s + NB, b).start()
            return carry

        lax.fori_loop(0, NI, loop_body, 0, unroll=False)

        for b in range(NB):
            write_copy(SEQLEN - NB + b, b).wait()

    return emb


_emb = _make_kernel()


@jax.jit
def kernel(inp, token_table, pos_table):
    idx4 = (
        inp.astype(jnp.int32)
        .T.reshape(SEQLEN // 8, 8, NW, BT)
        .transpose(0, 2, 1, 3)
    )
    pos4 = (
        jnp.pad(pos_table.T, ((0, 0), (0, 56)))
        .reshape(FT, 8, 2, BT)
        .transpose(0, 2, 1, 3)
    )
    out5 = _emb(token_table, idx4, pos4)
    return (
        out5.transpose(2, 4, 0, 1, 3)
        .reshape(BATCH, SEQLEN, EMBED)
    )


# speedup vs baseline: 1.0231x; 1.0010x over previous
import functools

import jax
import jax.numpy as jnp
from jax import lax
from jax.experimental import pallas as pl
from jax.experimental.pallas import tpu as pltpu
from jax.experimental.pallas import tpu_sc as plsc

VOCAB = 1000000
SEQLEN = 200
EMBED = 64
BATCH = 4096
LANES = 16

NW = 32
BT = BATCH // NW
FT = EMBED // 8


def _make_kernel():
    mesh = plsc.VectorSubcoreMesh(core_axis_name="c", subcore_axis_name="s")

    @functools.partial(
        pl.kernel,
        mesh=mesh,
        out_type=jax.ShapeDtypeStruct((SEQLEN, FT, NW, 8, BT), jnp.float32),
        compiler_params=pltpu.CompilerParams(
            use_tc_tiling_on_sc=False, needs_layout_passes=False
        ),
        scratch_types=[
            pltpu.VMEM((SEQLEN // 8, 8, BT), jnp.int32),
            pltpu.VMEM((FT, 2, 8, BT), jnp.float32),
            pltpu.VMEM((SEQLEN, EMBED), jnp.float32),
        ] + [pltpu.VMEM((BT, EMBED), jnp.float32) for _ in range(4)]
          + [pltpu.VMEM((FT, 8, BT), jnp.float32) for _ in range(4)]
          + [pltpu.SemaphoreType.DMA for _ in range(8)],
    )
    def emb(
        table_hbm, idx_hbm, pos_hbm, out_hbm,
        idx_v, pos_t_v, pos_v,
        rows0, rows1, rows2, rows3,
        slab0, slab1, slab2, slab3,
        gsem0, gsem1, gsem2, gsem3,
        wsem0, wsem1, wsem2, wsem3,
    ):
        w = lax.axis_index("s") * 2 + lax.axis_index("c")
        b0 = pl.multiple_of(w * BT, 8)

        rows = (rows0, rows1, rows2, rows3)
        slab = (slab0, slab1, slab2, slab3)
        gsem = (gsem0, gsem1, gsem2, gsem3)
        wsem = (wsem0, wsem1, wsem2, wsem3)

        pltpu.sync_copy(idx_hbm.at[:, w], idx_v)

        lane0 = lax.iota(jnp.int32, LANES)

        f_hi = [(q * LANES + lane0) >> 3 for q in range(EMBED // LANES)]
        f_lo = [(q * LANES + lane0) & 7 for q in range(EMBED // LANES)]

        def pos_body(s, _):
            st2 = jnp.broadcast_to(s >> 7, (LANES,))
            sc = jnp.broadcast_to(s & 127, (LANES,))
            for q in range(EMBED // LANES):
                vec = plsc.load_gather(pos_t_v, [f_hi[q], st2, f_lo[q], sc])
                pos_v[s, pl.ds(q * LANES, LANES)] = vec
            return 0

        def pos_prologue():
            pltpu.sync_copy(pos_hbm, pos_t_v)
            lax.fori_loop(0, SEQLEN, pos_body, 0)

        def gather_copy(s, b):
            return pltpu.make_async_copy(
                table_hbm.at[idx_v.at[s // 8, s % 8]], rows[b], gsem[b]
            )

        def write_copy(s, b):
            return pltpu.make_async_copy(
                slab[b], out_hbm.at[s, :, w], wsem[b]
            )

        lane = lax.iota(jnp.int32, LANES)
        row_ids = [lane + c * LANES for c in range(BT // LANES)]

        def transpose_add(s, b):
            pos_q = [pos_v[s, pl.ds(q * LANES, LANES)] for q in range(4)]

            def d_body(d, _):
                rot = (lane + d) & 15
                cols = [rot + q * LANES for q in range(4)]
                pos_d = [
                    pos_q[q].at[rot].get(mode="promise_in_bounds")
                    for q in range(4)
                ]
                vals = [
                    plsc.load_gather(rows[b], [row_ids[c], cols[q]])
                    for q in range(4)
                    for c in range(BT // LANES)
                ]
                sums = [
                    vals[q * (BT // LANES) + c] + pos_d[q]
                    for q in range(4)
                    for c in range(BT // LANES)
                ]
                for q in range(4):
                    for c in range(BT // LANES):
                        plsc.store_scatter(
                            slab[b],
                            [cols[q] >> 3, cols[q] & 7, row_ids[c]],
                            sums[q * (BT // LANES) + c],
                        )
                return 0

            lax.fori_loop(0, LANES, d_body, 0, unroll=4)

        NB = 4
        NI = SEQLEN // NB

        for b in range(NB):
            gather_copy(b, b).start()
        pos_prologue()

        def loop_body(i, carry):
            for b in range(NB):
                s = i * NB + b
                gather_copy(s, b).wait()

                @pl.when(i > 0)
                def _():
                    write_copy(s - NB, b).wait()
                transpose_add(s, b)
                write_copy(s, b).start()

                @pl.when(i < NI - 1)
                def _():
                    gather_copy(s + NB, b).start()
            return carry

        lax.fori_loop(0, NI, loop_body, 0, unroll=False)

        for b in range(NB):
            write_copy(SEQLEN - NB + b, b).wait()

    return emb


_emb = _make_kernel()


@jax.jit
def kernel(inp, token_table, pos_table):
    idx4 = (
        inp.astype(jnp.int32)
        .T.reshape(SEQLEN // 8, 8, NW, BT)
        .transpose(0, 2, 1, 3)
    )
    pos4 = (
        jnp.pad(pos_table.T, ((0, 0), (0, 56)))
        .reshape(FT, 8, 2, BT)
        .transpose(0, 2, 1, 3)
    )
    out5 = _emb(token_table, idx4, pos4)
    return (
        out5.transpose(2, 4, 0, 1, 3)
        .reshape(BATCH, SEQLEN, EMBED)
    )


# speedup vs baseline: 1.0231x; 1.0000x over previous
import functools

import jax
import jax.numpy as jnp
from jax import lax
from jax.experimental import pallas as pl
from jax.experimental.pallas import tpu as pltpu
from jax.experimental.pallas import tpu_sc as plsc

VOCAB = 1000000
SEQLEN = 200
EMBED = 64
BATCH = 4096
LANES = 16

NW = 32
BT = BATCH // NW
FT = EMBED // 8


def _make_kernel():
    mesh = plsc.VectorSubcoreMesh(core_axis_name="c", subcore_axis_name="s")

    @functools.partial(
        pl.kernel,
        mesh=mesh,
        out_type=jax.ShapeDtypeStruct((SEQLEN, FT, NW, 8, BT), jnp.float32),
        compiler_params=pltpu.CompilerParams(
            use_tc_tiling_on_sc=False, needs_layout_passes=False
        ),
        scratch_types=[
            pltpu.VMEM((SEQLEN // 8, 8, BT), jnp.int32),
            pltpu.VMEM((FT, 2, 8, BT), jnp.float32),
            pltpu.VMEM((SEQLEN, EMBED), jnp.float32),
        ] + [pltpu.VMEM((BT, EMBED), jnp.float32) for _ in range(4)]
          + [pltpu.VMEM((FT, 8, BT), jnp.float32) for _ in range(4)]
          + [pltpu.SemaphoreType.DMA for _ in range(8)],
    )
    def emb(
        table_hbm, idx_hbm, pos_hbm, out_hbm,
        idx_v, pos_t_v, pos_v,
        rows0, rows1, rows2, rows3,
        slab0, slab1, slab2, slab3,
        gsem0, gsem1, gsem2, gsem3,
        wsem0, wsem1, wsem2, wsem3,
    ):
        w = lax.axis_index("s") * 2 + lax.axis_index("c")

        rows = (rows0, rows1, rows2, rows3)
        slab = (slab0, slab1, slab2, slab3)
        gsem = (gsem0, gsem1, gsem2, gsem3)
        wsem = (wsem0, wsem1, wsem2, wsem3)

        pltpu.sync_copy(idx_hbm.at[:, w], idx_v)

        lane0 = lax.iota(jnp.int32, LANES)

        f_hi = [(q * LANES + lane0) >> 3 for q in range(EMBED // LANES)]
        f_lo = [(q * LANES + lane0) & 7 for q in range(EMBED // LANES)]

        def pos_body(s, _):
            st2 = jnp.broadcast_to(s >> 7, (LANES,))
            sc = jnp.broadcast_to(s & 127, (LANES,))
            for q in range(EMBED // LANES):
                vec = plsc.load_gather(pos_t_v, [f_hi[q], st2, f_lo[q], sc])
                pos_v[s, pl.ds(q * LANES, LANES)] = vec
            return 0

        def pos_prologue():
            pltpu.sync_copy(pos_hbm, pos_t_v)
            lax.fori_loop(0, SEQLEN, pos_body, 0)

        def gather_copy(s, b):
            return pltpu.make_async_copy(
                table_hbm.at[idx_v.at[s // 8, s % 8]], rows[b], gsem[b]
            )

        def write_copy(s, b):
            return pltpu.make_async_copy(
                slab[b], out_hbm.at[s, :, w], wsem[b]
            )

        lane = lax.iota(jnp.int32, LANES)
        row_ids = [lane + c * LANES for c in range(BT // LANES)]

        def transpose_add(s, b):
            pos_q = [pos_v[s, pl.ds(q * LANES, LANES)] for q in range(4)]

            def d_body(d, _):
                rot = (lane + d) & 15
                cols = [rot + q * LANES for q in range(4)]
                pos_d = [
                    pos_q[q].at[rot].get(mode="promise_in_bounds")
                    for q in range(4)
                ]
                vals = [
                    plsc.load_gather(rows[b], [row_ids[c], cols[q]])
                    for q in range(4)
                    for c in range(BT // LANES)
                ]
                sums = [
                    vals[q * (BT // LANES) + c] + pos_d[q]
                    for q in range(4)
                    for c in range(BT // LANES)
                ]
                for q in range(4):
                    for c in range(BT // LANES):
                        plsc.store_scatter(
                            slab[b],
                            [cols[q] >> 3, cols[q] & 7, row_ids[c]],
                            sums[q * (BT // LANES) + c],
                        )
                return 0

            lax.fori_loop(0, LANES, d_body, 0, unroll=4)

        NB = 4
        NI = SEQLEN // NB

        for b in range(NB):
            gather_copy(b, b).start()
        pos_prologue()

        def loop_body(i, carry):
            for b in range(NB):
                s = i * NB + b
                gather_copy(s, b).wait()

                @pl.when(i > 0)
                def _():
                    write_copy(s - NB, b).wait()
                transpose_add(s, b)
                write_copy(s, b).start()

                @pl.when(i < NI - 1)
                def _():
                    gather_copy(s + NB, b).start()
            return carry

        lax.fori_loop(0, NI, loop_body, 0, unroll=False)

        for b in range(NB):
            write_copy(SEQLEN - NB + b, b).wait()

    return emb


_emb = _make_kernel()


@jax.jit
def kernel(inp, token_table, pos_table):
    idx4 = (
        inp.astype(jnp.int32)
        .T.reshape(SEQLEN // 8, 8, NW, BT)
        .transpose(0, 2, 1, 3)
    )
    pos4 = (
        jnp.pad(pos_table.T, ((0, 0), (0, 56)))
        .reshape(FT, 8, 2, BT)
        .transpose(0, 2, 1, 3)
    )
    out5 = _emb(token_table, idx4, pos4)
    return (
        out5.transpose(2, 4, 0, 1, 3)
        .reshape(BATCH, SEQLEN, EMBED)
    )
